# baseline (speedup 1.0000x reference)
import functools

import jax
import jax.numpy as jnp
from jax import lax
from jax.experimental import pallas as pl
from jax.experimental.pallas import tpu as pltpu
from jax.experimental.pallas import tpu_sc as plsc

_NUM_EMB = 1000000
_DIM = 32
_MAX_NORM = 0.99
_BATCH = 16384

_NC = 2
_NS = 16
_L = 16
_NW = _NC * _NS
_BPW = _BATCH // _NW
_GROUPS = _BPW // _L


def _rsqrt(x):
    xhalf = x * jnp.float32(0.5)
    i = plsc.bitcast(x, jnp.int32)
    i = jnp.int32(0x5F3759DF) - lax.shift_right_logical(i, 1)
    y = plsc.bitcast(i, jnp.float32)
    for _ in range(3):
        y = y * (jnp.float32(1.5) - xhalf * y * y)
    return y


def _body(idx_hbm, table_hbm, out_hbm, idx_v, rows_v, sem):
    wid = lax.axis_index("s") * _NC + lax.axis_index("c")
    base = wid * _BPW
    pltpu.sync_copy(idx_hbm.at[pl.ds(base, _BPW)], idx_v)
    pltpu.async_copy(table_hbm.at[idx_v], rows_v, sem).wait()

    mn2 = jnp.float32(_MAX_NORM * _MAX_NORM)

    def group(g, carry):
        rows_idx = g * _L + lax.iota(jnp.int32, _L)
        acc = jnp.zeros((_L,), jnp.float32)
        cols = []
        for k in range(_DIM):
            col_idx = jnp.full((_L,), k, jnp.int32)
            v = plsc.load_gather(rows_v, [rows_idx, col_idx])
            cols.append(v)
            acc = acc + v * v
        scale = jnp.where(acc > mn2, jnp.float32(_MAX_NORM) * _rsqrt(acc),
                          jnp.float32(1.0))
        for k in range(_DIM):
            col_idx = jnp.full((_L,), k, jnp.int32)
            plsc.store_scatter(rows_v, [rows_idx, col_idx], cols[k] * scale)
        return carry

    lax.fori_loop(0, _GROUPS, group, 0)
    pltpu.sync_copy(rows_v, out_hbm.at[pl.ds(base, _BPW)])


_sc_call = functools.partial(
    pl.kernel,
    mesh=plsc.VectorSubcoreMesh(core_axis_name="c", subcore_axis_name="s"),
    out_type=jax.ShapeDtypeStruct((_BATCH, _DIM), jnp.float32),
    scratch_types=[
        pltpu.VMEM((_BPW,), jnp.int32),
        pltpu.VMEM((_BPW, _DIM), jnp.float32),
        pltpu.SemaphoreType.DMA,
    ],
    compiler_params=pltpu.CompilerParams(
        needs_layout_passes=False, use_tc_tiling_on_sc=False
    ),
)(_body)


def kernel(indices, embeddings):
    return _sc_call(indices.astype(jnp.int32), embeddings)


# speedup vs baseline: 3.9628x; 3.9628x over previous
import functools

import jax
import jax.numpy as jnp
from jax import lax
from jax.experimental import pallas as pl
from jax.experimental.pallas import tpu as pltpu
from jax.experimental.pallas import tpu_sc as plsc

_NUM_EMB = 1000000
_DIM = 32
_MAX_NORM = 0.99
_BATCH = 16384

_NC = 2
_NS = 16
_L = 16
_NW = _NC * _NS
_BPW = _BATCH // _NW
_NB = _BPW // 128
_G = 8


def _rsqrt(x):
    xhalf = x * jnp.float32(0.5)
    i = plsc.bitcast(x, jnp.int32)
    i = jnp.int32(0x5F3759DF) - lax.shift_right_logical(i, 1)
    y = plsc.bitcast(i, jnp.float32)
    for _ in range(3):
        y = y * (jnp.float32(1.5) - xhalf * y * y)
    return y


def _body(idx_hbm, tab_hbm, out_hbm, idx_s, rbufs, obufs, sems):
    wid = lax.axis_index("s") * _NC + lax.axis_index("c")
    base = wid * _BPW
    pltpu.sync_copy(idx_hbm.at[pl.ds(base, _BPW)], idx_s.at[pl.ds(0, _BPW)])

    r0 = lax.iota(jnp.int32, _L)
    r1 = r0 + _L
    mn2 = jnp.float32(_MAX_NORM * _MAX_NORM)

    def fire(i, slot):
        c = pl.multiple_of(lax.shift_left(lax.shift_right_logical(i, 7), 7),
                           128)
        pltpu.async_copy(tab_hbm.at[:, pl.ds(c, 128)], rbufs[slot],
                         sems[slot])

    def fire_set(idxv, lo, half):
        for b in range(_G):
            fire(idxv[half * _G + b], half * _G + b)

    def process(i, slot, col, obuf):
        pltpu.make_async_copy(tab_hbm.at[:, pl.ds(0, 128)], rbufs[slot],
                              sems[slot]).wait()
        lane = jnp.full((_L,), jnp.bitwise_and(i, jnp.int32(127)), jnp.int32)
        g0 = plsc.load_gather(rbufs[slot], [r0, lane])
        g1 = plsc.load_gather(rbufs[slot], [r1, lane])
        acc = jnp.sum(g0 * g0 + g1 * g1)
        accv = jnp.full((_L,), acc, jnp.float32)
        scale = jnp.where(accv > mn2, jnp.float32(_MAX_NORM) * _rsqrt(accv),
                          jnp.float32(1.0))
        colv = jnp.full((_L,), col, jnp.int32)
        plsc.store_scatter(obuf, [r0, colv], g0 * scale)
        plsc.store_scatter(obuf, [r1, colv], g1 * scale)

    for t in range(_NB):
        obuf = obufs[t]
        idxv0 = idx_s[pl.ds(t * 128, _L)]
        fire_set(idxv0, None, 0)

        def inner(p, carry, t=t, obuf=obuf):
            off = t * 128 + p * 16
            idxv = idx_s[pl.ds(off, _L)]
            idxn = idx_s[pl.ds(off + 16, _L)]
            fire_set(idxv, None, 1)
            for b in range(_G):
                process(idxv[b], b, p * 16 + b, obuf)

            @pl.when(p < (128 // 16) - 1)
            def _():
                fire_set(idxn, None, 0)

            for b in range(_G):
                process(idxv[_G + b], _G + b, p * 16 + _G + b, obuf)
            return carry

        lax.fori_loop(0, 128 // 16, inner, 0)
        pltpu.sync_copy(obuf, out_hbm.at[:, pl.ds(base + t * 128, 128)])


_sc_call = functools.partial(
    pl.kernel,
    mesh=plsc.VectorSubcoreMesh(core_axis_name="c", subcore_axis_name="s"),
    out_type=jax.ShapeDtypeStruct((_DIM, _BATCH), jnp.float32),
    scratch_types=[
        pltpu.VMEM((_BPW + _L,), jnp.int32),
        [pltpu.VMEM((_DIM, 128), jnp.float32)] * (2 * _G),
        [pltpu.VMEM((_DIM, 128), jnp.float32)] * _NB,
        [pltpu.SemaphoreType.DMA] * (2 * _G),
    ],
    compiler_params=pltpu.CompilerParams(needs_layout_passes=False),
)(_body)


def kernel(indices, embeddings):
    return _sc_call(indices.astype(jnp.int32), embeddings.T).T


# speedup vs baseline: 4.0278x; 1.0164x over previous
import functools

import jax
import jax.numpy as jnp
from jax import lax
from jax.experimental import pallas as pl
from jax.experimental.pallas import tpu as pltpu
from jax.experimental.pallas import tpu_sc as plsc

_NUM_EMB = 1000000
_DIM = 32
_MAX_NORM = 0.99
_BATCH = 16384

_NC = 2
_NS = 16
_L = 16
_NW = _NC * _NS
_BPW = _BATCH // _NW
_NB = _BPW // 128
_G = 8


def _rsqrt(x):
    xhalf = x * jnp.float32(0.5)
    i = plsc.bitcast(x, jnp.int32)
    i = jnp.int32(0x5F3759DF) - lax.shift_right_logical(i, 1)
    y = plsc.bitcast(i, jnp.float32)
    for _ in range(3):
        y = y * (jnp.float32(1.5) - xhalf * y * y)
    return y


def _body(idx_hbm, tab_hbm, out_hbm, idx_s, rbufs, obufs, sems, osem):
    wid = lax.axis_index("s") * _NC + lax.axis_index("c")
    base = wid * _BPW
    pltpu.sync_copy(idx_hbm.at[pl.ds(base, _BPW)], idx_s.at[pl.ds(0, _BPW)])

    r0 = lax.iota(jnp.int32, _L)
    r1 = r0 + _L
    mn2 = jnp.float32(_MAX_NORM * _MAX_NORM)

    def fire(i, slot):
        c = pl.multiple_of(lax.shift_left(lax.shift_right_logical(i, 7), 7),
                           128)
        pltpu.async_copy(tab_hbm.at[:, pl.ds(c, 128)], rbufs[slot],
                         sems[slot])

    def fire_set(idxv, lo, half):
        for b in range(_G):
            fire(idxv[half * _G + b], half * _G + b)

    def process(i, slot, col, obuf):
        pltpu.make_async_copy(tab_hbm.at[:, pl.ds(0, 128)], rbufs[slot],
                              sems[slot]).wait()
        lane = jnp.full((_L,), jnp.bitwise_and(i, jnp.int32(127)), jnp.int32)
        g0 = plsc.load_gather(rbufs[slot], [r0, lane])
        g1 = plsc.load_gather(rbufs[slot], [r1, lane])
        acc = jnp.sum(g0 * g0 + g1 * g1)
        accv = jnp.full((_L,), acc, jnp.float32)
        scale = jnp.where(accv > mn2, jnp.float32(_MAX_NORM) * _rsqrt(accv),
                          jnp.float32(1.0))
        colv = jnp.full((_L,), col, jnp.int32)
        plsc.store_scatter(obuf, [r0, colv], g0 * scale)
        plsc.store_scatter(obuf, [r1, colv], g1 * scale)

    idxv0 = idx_s[pl.ds(0, _L)]
    fire_set(idxv0, None, 0)

    for t in range(_NB):
        obuf = obufs[t]

        def inner(p, carry, t=t, obuf=obuf):
            off = t * 128 + p * 16
            idxv = idx_s[pl.ds(off, _L)]
            idxn = idx_s[pl.ds(off + 16, _L)]
            fire_set(idxv, None, 1)
            for b in range(_G):
                process(idxv[b], b, p * 16 + b, obuf)

            if t < _NB - 1:
                fire_set(idxn, None, 0)
            else:

                @pl.when(p < (128 // 16) - 1)
                def _():
                    fire_set(idxn, None, 0)

            for b in range(_G):
                process(idxv[_G + b], _G + b, p * 16 + _G + b, obuf)
            return carry

        lax.fori_loop(0, 128 // 16, inner, 0)
        pltpu.async_copy(obuf, out_hbm.at[:, pl.ds(base + t * 128, 128)],
                         osem)

    for t in range(_NB):
        pltpu.make_async_copy(
            obufs[t], out_hbm.at[:, pl.ds(base + t * 128, 128)], osem
        ).wait()


_sc_call = functools.partial(
    pl.kernel,
    mesh=plsc.VectorSubcoreMesh(core_axis_name="c", subcore_axis_name="s"),
    out_type=jax.ShapeDtypeStruct((_DIM, _BATCH), jnp.float32),
    scratch_types=[
        pltpu.VMEM((_BPW + _L,), jnp.int32),
        [pltpu.VMEM((_DIM, 128), jnp.float32)] * (2 * _G),
        [pltpu.VMEM((_DIM, 128), jnp.float32)] * _NB,
        [pltpu.SemaphoreType.DMA] * (2 * _G),
        pltpu.SemaphoreType.DMA,
    ],
    compiler_params=pltpu.CompilerParams(needs_layout_passes=False),
)(_body)


def kernel(indices, embeddings):
    return _sc_call(indices.astype(jnp.int32), embeddings.T).T
